# baseline (speedup 1.0000x reference)
import functools

import jax
import jax.numpy as jnp
from jax import lax
from jax.experimental import pallas as pl
from jax.experimental.pallas import tpu as pltpu
from jax.experimental.pallas import tpu_sc as plsc

N = 10000
E = 320000
D_FEAT = 128
D_EDGE = 16
MSG = 64
HID = 128

NC = 2
NS = 16
NW = NC * NS
CHUNK = 128
K_CHUNKS = 79
EPW = CHUNK * K_CHUNKS
E_PAD = NW * EPW
N_PAD = 10240
EBLK = 2048


def _node_proj_body(h_ref, ws_ref, wd_ref, a_ref, b_ref):
    h = h_ref[...]
    a_ref[...] = jnp.dot(h, ws_ref[...],
                         preferred_element_type=jnp.float32).astype(
                             jnp.bfloat16)
    b_ref[...] = jnp.dot(h, wd_ref[...],
                         preferred_element_type=jnp.float32).astype(
                             jnp.bfloat16)


_node_proj = pl.pallas_call(
    _node_proj_body,
    out_shape=[
        jax.ShapeDtypeStruct((N_PAD, MSG), jnp.bfloat16),
        jax.ShapeDtypeStruct((N_PAD, MSG), jnp.bfloat16),
    ],
)


def _edge_proj_body(e_ref, w_ref, b_ref, c_ref):
    x = e_ref[...]
    w = w_ref[...]
    b = b_ref[...]
    top = jnp.dot(x[:1024], w, preferred_element_type=jnp.float32) + b
    bot = jnp.dot(x[1024:], w, preferred_element_type=jnp.float32) + b
    c_ref[...] = jnp.concatenate([top, bot], axis=1).astype(jnp.bfloat16)


_edge_proj = pl.pallas_call(
    _edge_proj_body,
    grid=(E + 2047) // 2048,
    in_specs=[
        pl.BlockSpec((2048, D_EDGE), lambda i: (i, 0)),
        pl.BlockSpec((D_EDGE, MSG), lambda i: (0, 0)),
        pl.BlockSpec((1, MSG), lambda i: (0, 0)),
    ],
    out_specs=pl.BlockSpec((1024, 128), lambda i: (i, 0)),
    out_shape=jax.ShapeDtypeStruct((E_PAD // 2, 128), jnp.bfloat16),
)


def _update_body(p_ref, h_ref, wt_ref, wb_ref, bh_ref, o_ref):
    m_sum = p_ref[0, :N] + p_ref[1, :N]
    o_ref[...] = jnp.maximum(
        jnp.dot(m_sum, wt_ref[...], preferred_element_type=jnp.float32)
        + jnp.dot(h_ref[...], wb_ref[...], preferred_element_type=jnp.float32)
        + bh_ref[...],
        0.0,
    )


_update = pl.pallas_call(
    _update_body,
    out_shape=jax.ShapeDtypeStruct((N, HID), jnp.float32),
)


_mesh = plsc.VectorSubcoreMesh(core_axis_name="c", subcore_axis_name="s")


@functools.partial(
    pl.kernel,
    out_type=jax.ShapeDtypeStruct((NC, N_PAD, MSG), jnp.float32),
    mesh=_mesh,
    compiler_params=pltpu.CompilerParams(use_tc_tiling_on_sc=False,
                                        needs_layout_passes=False),
    scratch_types=[
        pltpu.VMEM((K_CHUNKS, CHUNK), jnp.int32),
        pltpu.VMEM((K_CHUNKS, CHUNK), jnp.int32),
        pltpu.VMEM((CHUNK, MSG), jnp.bfloat16),
        pltpu.VMEM((CHUNK, MSG), jnp.bfloat16),
        pltpu.VMEM((CHUNK, MSG), jnp.bfloat16),
        pltpu.VMEM((CHUNK, MSG), jnp.bfloat16),
        pltpu.VMEM((CHUNK, MSG), jnp.bfloat16),
        pltpu.VMEM((CHUNK, MSG), jnp.bfloat16),
        pltpu.VMEM((CHUNK, MSG), jnp.float32),
        pltpu.VMEM((CHUNK, MSG), jnp.float32),
        pltpu.VMEM_SHARED((N_PAD, MSG), jnp.float32),
        pltpu.SemaphoreType.DMA,
        pltpu.SemaphoreType.DMA,
        pltpu.SemaphoreType.DMA,
        pltpu.SemaphoreType.DMA,
    ],
)
def _edge_phase(a_hbm, b_hbm, c_hbm, src_hbm, dst_hbm, out_hbm,
                src_v, dst_v, a0, a1, b0, b1, c0, c1,
                m0, m1, acc, si0, si1, ss0, ss1):
    cid = lax.axis_index("c")
    sid = lax.axis_index("s")
    wid = sid * NC + cid
    abuf = (a0, a1)
    bbuf = (b0, b1)
    cbuf = (c0, c1)
    mbuf = (m0, m1)
    si = (si0, si1)
    ss = (ss0, ss1)
    dummy = a_hbm.at[pl.ds(0, CHUNK)]
    dummy_f = out_hbm.at[0, pl.ds(0, CHUNK)]

    zero16 = jnp.zeros((16,), jnp.float32)

    def _zrow(r, _):
        for j in range(MSG // 16):
            m0[r, pl.ds(j * 16, 16)] = zero16
        return 0

    lax.fori_loop(0, CHUNK, _zrow, 0)
    for j in range(5):
        pltpu.sync_copy(m0, acc.at[pl.ds(sid * 640 + j * CHUNK, CHUNK)])
    plsc.subcore_barrier()

    pltpu.sync_copy(src_hbm.at[wid], src_v)
    pltpu.sync_copy(dst_hbm.at[wid], dst_v)

    def fire(k, par, slot):
        @pl.when(k >= 2)
        def _():
            pltpu.make_async_copy(dummy_f, mbuf[slot], ss[slot]).wait()

        pos = (k * NW + wid) * CHUNK
        row0 = lax.shift_right_logical(pos, 11) * 1024 + (pos & 1023)
        col0 = (lax.shift_right_logical(pos, 10) & 1) * MSG
        pltpu.async_copy(a_hbm.at[src_v.at[k]], abuf[par], si[par])
        pltpu.async_copy(b_hbm.at[dst_v.at[k]], bbuf[par], si[par])
        pltpu.async_copy(c_hbm.at[pl.ds(row0, CHUNK), pl.ds(col0, MSG)],
                         cbuf[slot], si[par])

    def proc(k, par, slot):
        a_v, b_v, c_v, m_v = abuf[par], bbuf[par], cbuf[slot], mbuf[slot]
        pltpu.make_async_copy(dummy, a_v, si[par]).wait()
        pltpu.make_async_copy(dummy, b_v, si[par]).wait()
        pltpu.make_async_copy(dummy, c_v, si[par]).wait()

        zero32 = jnp.zeros((32,), jnp.bfloat16)

        def _row(r4, _):
            for rr in range(4):
                r = r4 * 4 + rr
                for g in range(MSG // 32):
                    sl = pl.ds(g * 32, 32)
                    msg = jnp.maximum(a_v[r, sl] + b_v[r, sl] + c_v[r, sl],
                                      zero32)
                    lo, hi = plsc.unpack(msg,
                                         format=plsc.PackFormat.INTERLEAVED)
                    m_v[r, pl.ds(g * 32, 16)] = lo
                    m_v[r, pl.ds(g * 32 + 16, 16)] = hi
            return 0

        lax.fori_loop(0, CHUNK // 4, _row, 0)
        pltpu.async_copy(m_v, acc.at[dst_v.at[k]], ss[slot], add=True)

    fire(0, 0, 0)

    def _pair(i, _):
        k = 2 * i
        fire(k + 1, 1, 1)
        proc(k, 0, 0)
        fire(k + 2, 0, 0)
        proc(k + 1, 1, 1)
        return 0

    lax.fori_loop(0, (K_CHUNKS - 1) // 2, _pair, 0)
    proc(78, 0, 0)
    for j in range(2):
        pltpu.make_async_copy(dummy_f, mbuf[j], ss[j]).wait()
    plsc.subcore_barrier()

    pltpu.sync_copy(acc.at[pl.ds(sid * 640, 640)],
                    out_hbm.at[cid, pl.ds(sid * 640, 640)])


def kernel(x, edge_index, edge_attr, node_ids,
           Wm0, bm0, Wh0, bh0, Wm1, bm1, Wh1, bh1):
    del node_ids
    pad_e = E_PAD - E

    def _split(v):
        return v.reshape(K_CHUNKS, NW, CHUNK).transpose(1, 0, 2)

    src_p = _split(jnp.concatenate(
        [edge_index[0], jnp.zeros((pad_e,), jnp.int32)]))
    trash = N + (jnp.arange(pad_e, dtype=jnp.int32) % (N_PAD - N))
    dst_p = _split(jnp.concatenate([edge_index[1], trash]))

    h = x
    for Wm, bm, Wh, bh in ((Wm0, bm0, Wh0, bh0), (Wm1, bm1, Wh1, bh1)):
        d = h.shape[1]
        h_pad = jnp.concatenate([h, jnp.zeros((N_PAD - N, d), jnp.float32)])
        a_t, b_t = _node_proj(h_pad, Wm[:d], Wm[d:2 * d])
        c_t = _edge_proj(edge_attr, Wm[2 * d:], bm.reshape(1, MSG))
        parts = _edge_phase(a_t, b_t, c_t, src_p, dst_p)
        p64 = jnp.arange(MSG)
        srcdim = 32 * (p64 // 32) + 2 * (p64 % 16) + ((p64 % 32) // 16)
        h = _update(parts, h, Wh[:MSG][srcdim], Wh[MSG:], bh.reshape(1, HID))
    return h


# speedup vs baseline: 1.2917x; 1.2917x over previous
import functools

import jax
import jax.numpy as jnp
from jax import lax
from jax.experimental import pallas as pl
from jax.experimental.pallas import tpu as pltpu
from jax.experimental.pallas import tpu_sc as plsc

N = 10000
E = 320000
D_FEAT = 128
D_EDGE = 16
MSG = 64
HID = 128

NC = 2
NS = 16
NW = NC * NS
CHUNK = 128
K_CHUNKS = 79
EPW = CHUNK * K_CHUNKS
E_PAD = NW * EPW
N_PAD = 10240
EBLK = 2048


def _node_proj_body(h_ref, ws_ref, wd_ref, a_ref, b_ref):
    h = h_ref[...]
    a_ref[...] = jnp.dot(h, ws_ref[...], preferred_element_type=jnp.float32)
    b_ref[...] = jnp.dot(h, wd_ref[...], preferred_element_type=jnp.float32)


_node_proj = pl.pallas_call(
    _node_proj_body,
    out_shape=[
        jax.ShapeDtypeStruct((N_PAD, MSG), jnp.float32),
        jax.ShapeDtypeStruct((N_PAD, MSG), jnp.float32),
    ],
)


def _edge_proj_body(e_ref, w_ref, b_ref, c_ref):
    x = e_ref[...]
    w = w_ref[...]
    b = b_ref[...]
    top = jnp.dot(x[:1024], w, preferred_element_type=jnp.float32) + b
    bot = jnp.dot(x[1024:], w, preferred_element_type=jnp.float32) + b
    c_ref[...] = jnp.concatenate([top, bot], axis=1)


_edge_proj = pl.pallas_call(
    _edge_proj_body,
    grid=(E + 2047) // 2048,
    in_specs=[
        pl.BlockSpec((2048, D_EDGE), lambda i: (i, 0)),
        pl.BlockSpec((D_EDGE, MSG), lambda i: (0, 0)),
        pl.BlockSpec((1, MSG), lambda i: (0, 0)),
    ],
    out_specs=pl.BlockSpec((1024, 128), lambda i: (i, 0)),
    out_shape=jax.ShapeDtypeStruct((E_PAD // 2, 128), jnp.float32),
)


def _update_body(p_ref, h_ref, wt_ref, wb_ref, bh_ref, o_ref):
    m_sum = p_ref[0, :N] + p_ref[1, :N]
    o_ref[...] = jnp.maximum(
        jnp.dot(m_sum, wt_ref[...], preferred_element_type=jnp.float32)
        + jnp.dot(h_ref[...], wb_ref[...], preferred_element_type=jnp.float32)
        + bh_ref[...],
        0.0,
    )


_update = pl.pallas_call(
    _update_body,
    out_shape=jax.ShapeDtypeStruct((N, HID), jnp.float32),
)


_mesh = plsc.VectorSubcoreMesh(core_axis_name="c", subcore_axis_name="s")


@functools.partial(
    pl.kernel,
    out_type=jax.ShapeDtypeStruct((NC, N_PAD, MSG), jnp.float32),
    mesh=_mesh,
    compiler_params=pltpu.CompilerParams(use_tc_tiling_on_sc=False),
    scratch_types=[
        pltpu.VMEM((K_CHUNKS, CHUNK), jnp.int32),
        pltpu.VMEM((K_CHUNKS, CHUNK), jnp.int32),
        pltpu.VMEM((CHUNK, MSG), jnp.float32),
        pltpu.VMEM((CHUNK, MSG), jnp.float32),
        pltpu.VMEM((CHUNK, MSG), jnp.float32),
        pltpu.VMEM((CHUNK, MSG), jnp.float32),
        pltpu.VMEM((CHUNK, MSG), jnp.float32),
        pltpu.VMEM((CHUNK, MSG), jnp.float32),
        pltpu.VMEM_SHARED((N_PAD, MSG), jnp.float32),
        pltpu.SemaphoreType.DMA,
        pltpu.SemaphoreType.DMA,
        pltpu.SemaphoreType.DMA,
        pltpu.SemaphoreType.DMA,
    ],
)
def _edge_phase(a_hbm, b_hbm, c_hbm, src_hbm, dst_hbm, out_hbm,
                src_v, dst_v, a0, a1, b0, b1,
                m0, m1, acc, si0, si1, ss0, ss1):
    cid = lax.axis_index("c")
    sid = lax.axis_index("s")
    wid = sid * NC + cid
    abuf = (a0, a1)
    bbuf = (b0, b1)
    mbuf = (m0, m1)
    si = (si0, si1)
    ss = (ss0, ss1)
    dummy = a_hbm.at[pl.ds(0, CHUNK)]

    zero16 = jnp.zeros((16,), jnp.float32)

    def _zrow(r, _):
        for j in range(MSG // 16):
            m0[r, pl.ds(j * 16, 16)] = zero16
        return 0

    lax.fori_loop(0, CHUNK, _zrow, 0)
    for j in range(5):
        pltpu.sync_copy(m0, acc.at[pl.ds(sid * 640 + j * CHUNK, CHUNK)])
    plsc.subcore_barrier()

    pltpu.sync_copy(src_hbm.at[wid], src_v)
    pltpu.sync_copy(dst_hbm.at[wid], dst_v)

    def fire(k, par, slot):
        @pl.when(k >= 2)
        def _():
            pltpu.make_async_copy(dummy, mbuf[slot], ss[slot]).wait()

        pos = (k * NW + wid) * CHUNK
        row0 = lax.shift_right_logical(pos, 11) * 1024 + (pos & 1023)
        col0 = (lax.shift_right_logical(pos, 10) & 1) * MSG
        pltpu.async_copy(a_hbm.at[src_v.at[k]], abuf[par], si[par])
        pltpu.async_copy(b_hbm.at[dst_v.at[k]], bbuf[par], si[par])
        pltpu.async_copy(c_hbm.at[pl.ds(row0, CHUNK), pl.ds(col0, MSG)],
                         mbuf[slot], si[par])

    def proc(k, par, slot):
        a_v, b_v, m_v = abuf[par], bbuf[par], mbuf[slot]
        pltpu.make_async_copy(dummy, a_v, si[par]).wait()
        pltpu.make_async_copy(dummy, b_v, si[par]).wait()
        pltpu.make_async_copy(dummy, m_v, si[par]).wait()

        def _row(r4, _):
            for rr in range(4):
                r = r4 * 4 + rr
                for j in range(MSG // 16):
                    sl = pl.ds(j * 16, 16)
                    m_v[r, sl] = jnp.maximum(
                        a_v[r, sl] + b_v[r, sl] + m_v[r, sl], 0.0)
            return 0

        lax.fori_loop(0, CHUNK // 4, _row, 0)
        pltpu.async_copy(m_v, acc.at[dst_v.at[k]], ss[slot], add=True)

    fire(0, 0, 0)

    def _pair(i, _):
        k = 2 * i
        fire(k + 1, 1, 1)
        proc(k, 0, 0)
        fire(k + 2, 0, 0)
        proc(k + 1, 1, 1)
        return 0

    lax.fori_loop(0, (K_CHUNKS - 1) // 2, _pair, 0)
    proc(78, 0, 0)
    for j in range(2):
        pltpu.make_async_copy(dummy, mbuf[j], ss[j]).wait()
    plsc.subcore_barrier()

    pltpu.sync_copy(acc.at[pl.ds(sid * 640, 640)],
                    out_hbm.at[cid, pl.ds(sid * 640, 640)])


def kernel(x, edge_index, edge_attr, node_ids,
           Wm0, bm0, Wh0, bh0, Wm1, bm1, Wh1, bh1):
    del node_ids
    pad_e = E_PAD - E

    def _split(v):
        return v.reshape(K_CHUNKS, NW, CHUNK).transpose(1, 0, 2)

    src_p = _split(jnp.concatenate(
        [edge_index[0], jnp.zeros((pad_e,), jnp.int32)]))
    trash = N + (jnp.arange(pad_e, dtype=jnp.int32) % (N_PAD - N))
    dst_p = _split(jnp.concatenate([edge_index[1], trash]))

    h = x
    for Wm, bm, Wh, bh in ((Wm0, bm0, Wh0, bh0), (Wm1, bm1, Wh1, bh1)):
        d = h.shape[1]
        h_pad = jnp.concatenate([h, jnp.zeros((N_PAD - N, d), jnp.float32)])
        a_t, b_t = _node_proj(h_pad, Wm[:d], Wm[d:2 * d])
        c_t = _edge_proj(edge_attr, Wm[2 * d:], bm.reshape(1, MSG))
        parts = _edge_phase(a_t, b_t, c_t, src_p, dst_p)
        h = _update(parts, h, Wh[:MSG], Wh[MSG:], bh.reshape(1, HID))
    return h


# speedup vs baseline: 1.3758x; 1.0652x over previous
import functools

import jax
import jax.numpy as jnp
from jax import lax
from jax.experimental import pallas as pl
from jax.experimental.pallas import tpu as pltpu
from jax.experimental.pallas import tpu_sc as plsc

N = 10000
E = 320000
D_FEAT = 128
D_EDGE = 16
MSG = 64
HID = 128

NC = 2
NS = 16
NW = NC * NS
CHUNK = 128
K_CHUNKS = 79
EPW = CHUNK * K_CHUNKS
E_PAD = NW * EPW
N_PAD = 10240
EBLK = 2048


def _node_proj_body(h_ref, ws_ref, wd_ref, a_ref, b_ref):
    h = h_ref[...]
    a_ref[...] = jnp.dot(h, ws_ref[...], preferred_element_type=jnp.float32)
    b_ref[...] = jnp.dot(h, wd_ref[...], preferred_element_type=jnp.float32)


_node_proj = pl.pallas_call(
    _node_proj_body,
    out_shape=[
        jax.ShapeDtypeStruct((N_PAD, MSG), jnp.float32),
        jax.ShapeDtypeStruct((N_PAD, MSG), jnp.float32),
    ],
)


def _edge_proj_body(e_ref, w_ref, b_ref, c_ref):
    x = e_ref[...]
    w = w_ref[...]
    b = b_ref[...]
    top = jnp.dot(x[:2048], w, preferred_element_type=jnp.float32) + b
    bot = jnp.dot(x[2048:], w, preferred_element_type=jnp.float32) + b
    c_ref[...] = jnp.concatenate([top, bot], axis=1)


_edge_proj = pl.pallas_call(
    _edge_proj_body,
    grid=E_PAD // 4096,
    in_specs=[
        pl.BlockSpec((4096, D_EDGE), lambda i: (i, 0)),
        pl.BlockSpec((D_EDGE, MSG), lambda i: (0, 0)),
        pl.BlockSpec((1, MSG), lambda i: (0, 0)),
    ],
    out_specs=pl.BlockSpec((2048, 128), lambda i: (i, 0)),
    out_shape=jax.ShapeDtypeStruct((E_PAD // 2, 128), jnp.float32),
)


def _update_body(p_ref, h_ref, wt_ref, wb_ref, bh_ref, o_ref):
    m_sum = p_ref[0, :N] + p_ref[1, :N]
    o_ref[...] = jnp.maximum(
        jnp.dot(m_sum, wt_ref[...], preferred_element_type=jnp.float32)
        + jnp.dot(h_ref[...], wb_ref[...], preferred_element_type=jnp.float32)
        + bh_ref[...],
        0.0,
    )


_update = pl.pallas_call(
    _update_body,
    out_shape=jax.ShapeDtypeStruct((N, HID), jnp.float32),
)


_mesh = plsc.VectorSubcoreMesh(core_axis_name="c", subcore_axis_name="s")


@functools.partial(
    pl.kernel,
    out_type=jax.ShapeDtypeStruct((NC, N_PAD, MSG), jnp.float32),
    mesh=_mesh,
    compiler_params=pltpu.CompilerParams(use_tc_tiling_on_sc=False),
    scratch_types=[
        pltpu.VMEM((K_CHUNKS, CHUNK), jnp.int32),
        pltpu.VMEM((K_CHUNKS, CHUNK), jnp.int32),
        pltpu.VMEM((CHUNK, MSG), jnp.float32),
        pltpu.VMEM((CHUNK, MSG), jnp.float32),
        pltpu.VMEM((CHUNK, MSG), jnp.float32),
        pltpu.VMEM((CHUNK, MSG), jnp.float32),
        pltpu.VMEM((CHUNK, MSG), jnp.float32),
        pltpu.VMEM((CHUNK, MSG), jnp.float32),
        pltpu.VMEM_SHARED((N_PAD, MSG), jnp.float32),
        pltpu.SemaphoreType.DMA,
        pltpu.SemaphoreType.DMA,
        pltpu.SemaphoreType.DMA,
        pltpu.SemaphoreType.DMA,
    ],
)
def _edge_phase(a_hbm, b_hbm, c_hbm, src_hbm, dst_hbm, out_hbm,
                src_v, dst_v, a0, a1, b0, b1,
                m0, m1, acc, si0, si1, ss0, ss1):
    cid = lax.axis_index("c")
    sid = lax.axis_index("s")
    wid = sid * NC + cid
    abuf = (a0, a1)
    bbuf = (b0, b1)
    mbuf = (m0, m1)
    si = (si0, si1)
    ss = (ss0, ss1)
    dummy = a_hbm.at[pl.ds(0, CHUNK)]

    zero16 = jnp.zeros((16,), jnp.float32)

    def _zrow(r, _):
        for j in range(MSG // 16):
            m0[r, pl.ds(j * 16, 16)] = zero16
        return 0

    lax.fori_loop(0, CHUNK, _zrow, 0)
    for j in range(5):
        pltpu.sync_copy(m0, acc.at[pl.ds(sid * 640 + j * CHUNK, CHUNK)])
    plsc.subcore_barrier()

    pltpu.sync_copy(src_hbm.at[wid], src_v)
    pltpu.sync_copy(dst_hbm.at[wid], dst_v)

    def fire(k, par, slot):
        @pl.when(k >= 2)
        def _():
            pltpu.make_async_copy(dummy, mbuf[slot], ss[slot]).wait()

        pos = (k * NW + wid) * CHUNK
        row0 = lax.shift_right_logical(pos, 12) * 2048 + (pos & 2047)
        col0 = (lax.shift_right_logical(pos, 11) & 1) * MSG
        pltpu.async_copy(a_hbm.at[src_v.at[k]], abuf[par], si[par])
        pltpu.async_copy(b_hbm.at[dst_v.at[k]], bbuf[par], si[par])
        pltpu.async_copy(c_hbm.at[pl.ds(row0, CHUNK), pl.ds(col0, MSG)],
                         mbuf[slot], si[par])

    def proc(k, par, slot):
        a_v, b_v, m_v = abuf[par], bbuf[par], mbuf[slot]
        pltpu.make_async_copy(dummy, a_v, si[par]).wait()
        pltpu.make_async_copy(dummy, b_v, si[par]).wait()
        pltpu.make_async_copy(dummy, m_v, si[par]).wait()

        def _row(r4, _):
            for rr in range(4):
                r = r4 * 4 + rr
                for j in range(MSG // 16):
                    sl = pl.ds(j * 16, 16)
                    m_v[r, sl] = jnp.maximum(
                        a_v[r, sl] + b_v[r, sl] + m_v[r, sl], 0.0)
            return 0

        lax.fori_loop(0, CHUNK // 4, _row, 0)
        pltpu.async_copy(m_v, acc.at[dst_v.at[k]], ss[slot], add=True)

    fire(0, 0, 0)

    def _pair(i, _):
        k = 2 * i
        fire(k + 1, 1, 1)
        proc(k, 0, 0)
        fire(k + 2, 0, 0)
        proc(k + 1, 1, 1)
        return 0

    lax.fori_loop(0, (K_CHUNKS - 1) // 2, _pair, 0)
    proc(78, 0, 0)
    for j in range(2):
        pltpu.make_async_copy(dummy, mbuf[j], ss[j]).wait()
    plsc.subcore_barrier()

    pltpu.sync_copy(acc.at[pl.ds(sid * 640, 640)],
                    out_hbm.at[cid, pl.ds(sid * 640, 640)])


def kernel(x, edge_index, edge_attr, node_ids,
           Wm0, bm0, Wh0, bh0, Wm1, bm1, Wh1, bh1):
    del node_ids
    pad_e = E_PAD - E

    def _split(v):
        return v.reshape(K_CHUNKS, NW, CHUNK).transpose(1, 0, 2)

    src_p = _split(jnp.concatenate(
        [edge_index[0], jnp.zeros((pad_e,), jnp.int32)]))
    trash = N + (jnp.arange(pad_e, dtype=jnp.int32) % (N_PAD - N))
    dst_p = _split(jnp.concatenate([edge_index[1], trash]))

    h = x
    for Wm, bm, Wh, bh in ((Wm0, bm0, Wh0, bh0), (Wm1, bm1, Wh1, bh1)):
        d = h.shape[1]
        h_pad = jnp.concatenate([h, jnp.zeros((N_PAD - N, d), jnp.float32)])
        a_t, b_t = _node_proj(h_pad, Wm[:d], Wm[d:2 * d])
        c_t = _edge_proj(edge_attr, Wm[2 * d:], bm.reshape(1, MSG))
        parts = _edge_phase(a_t, b_t, c_t, src_p, dst_p)
        h = _update(parts, h, Wh[:MSG], Wh[MSG:], bh.reshape(1, HID))
    return h
